# baseline (speedup 1.0000x reference)
import jax
import jax.numpy as jnp
from jax import lax
from jax.experimental import pallas as pl
from jax.experimental.pallas import tpu as pltpu
from jax.experimental.pallas import tpu_sc as plsc

NS = 16
NC = 2
CC = 4
NBUF = 5
CW_CNT = 16


def _pad_rows(n):
    g = NS * 8
    return (n + g - 1) // g * g


def _sc_gather_segsum(x_user, x_item, su, du, si, di, chunk, zrows,
                      interpret=False):
    n_user, d = x_user.shape
    n_item = x_item.shape[0]
    e = su.shape[0]
    cw = d // CC
    eper = e // NS
    steps = eper // (chunk * NBUF)
    assert eper % (chunk * NBUF) == 0 and chunk % 16 == 0 and chunk <= 128
    np_u, np_i = _pad_rows(n_user), _pad_rows(n_item)
    assert zrows % 8 == 0

    xu_flat = x_user.reshape(n_user * CC, cw)
    xi_flat = x_item.reshape(n_item * CC, cw)
    ones_h = jnp.ones((chunk, cw), jnp.float32)
    zeros_h = jnp.zeros((zrows, cw), jnp.float32)

    mesh = plsc.VectorSubcoreMesh(core_axis_name="c", subcore_axis_name="s",
                                  num_cores=NC, num_subcores=NS)

    def body(xu_h, xi_h, su_h, du_h, si_h, di_h, ones_hb, zeros_hb,
             o_sum_item, o_sum_user, o_cnt_item, o_cnt_user,
             acc, sidx, didx, rows, ones_v,
             isem, dsem, gsem, asem):
        cid = lax.axis_index("c")
        sid = lax.axis_index("s")
        pltpu.sync_copy(ones_hb, ones_v)
        base = sid * eper

        def zero_acc(npad):
            npt = npad // NS
            full, tail = npt // zrows, npt % zrows
            for j in range(full):
                pltpu.sync_copy(zeros_hb,
                                acc.at[pl.ds(sid * npt + j * zrows, zrows)])
            if tail:
                pltpu.sync_copy(zeros_hb.at[pl.ds(0, tail)],
                                acc.at[pl.ds(sid * npt + full * zrows, tail)])

        def scan_edges(dst_h, src_h=None, table=None, c=0):
            def issue_idx(i, b):
                off = pl.multiple_of(base + (i * NBUF + b) * chunk, 8)
                dd = pltpu.async_copy(dst_h.at[pl.ds(off, chunk)],
                                      didx.at[b], dsem.at[b])
                sd = None
                if src_h is not None:
                    sd = pltpu.async_copy(src_h.at[pl.ds(off, chunk)],
                                          sidx.at[b], isem.at[b])
                return dd, sd

            def gather(b, sd):
                sd.wait()
                for j in range(chunk // 16):
                    s16 = sidx[b, pl.ds(16 * j, 16)]
                    sidx[b, pl.ds(16 * j, 16)] = s16 * CC + c
                return pltpu.async_copy(table.at[sidx.at[b]],
                                        rows.at[b], gsem.at[b])

            def scatter(b, dd, gd):
                dd.wait()
                if src_h is not None:
                    gd.wait()
                    return pltpu.async_copy(rows.at[b], acc.at[didx.at[b]],
                                            asem.at[b], add=True)
                return pltpu.async_copy(ones_v, acc.at[didx.at[b]],
                                        asem.at[b], add=True)

            def drain_scatter(b):
                if src_h is not None:
                    pltpu.make_async_copy(rows.at[b], acc.at[didx.at[b]],
                                          asem.at[b]).wait()
                else:
                    pltpu.make_async_copy(ones_v, acc.at[didx.at[b]],
                                          asem.at[b]).wait()

            def run_step(i, drain_first):
                dd = [None] * NBUF
                sd = [None] * NBUF
                gd = [None] * NBUF
                for b in range(NBUF):
                    if drain_first:
                        drain_scatter(b)
                    dd[b], sd[b] = issue_idx(i, b)
                if src_h is not None:
                    for b in range(NBUF):
                        gd[b] = gather(b, sd[b])
                for b in range(NBUF):
                    scatter(b, dd[b], gd[b])

            run_step(0, drain_first=False)

            def step(i, carry):
                run_step(i, drain_first=True)
                return carry
            lax.fori_loop(1, steps, step, 0)
            for b in range(NBUF):
                drain_scatter(b)

        def flush(npad, dst):
            npt = npad // NS
            pltpu.sync_copy(acc.at[pl.ds(sid * npt, npt)], dst)

        def do_counts(dst_h, npad, out_ref):
            zero_acc(npad)
            plsc.subcore_barrier()
            scan_edges(dst_h)
            plsc.subcore_barrier()
            npt = npad // NS
            pltpu.sync_copy(acc.at[pl.ds(sid * npt, npt), pl.ds(0, CW_CNT)],
                            out_ref.at[pl.ds(sid * npt, npt)])
            plsc.subcore_barrier()

        def do_task(src_h, dst_h, table, npad, out_ref, c):
            zero_acc(npad)
            plsc.subcore_barrier()
            scan_edges(dst_h, src_h, table, c)
            plsc.subcore_barrier()
            npt = npad // NS
            flush(npad, out_ref.at[pl.ds(sid * npt, npt), pl.ds(cw * c, cw)])
            plsc.subcore_barrier()

        @pl.when(cid == 0)
        def _():
            do_counts(di_h, np_u, o_cnt_user)
            do_task(su_h, du_h, xu_h, np_i, o_sum_item, 0)
            do_task(su_h, du_h, xu_h, np_i, o_sum_item, 1)
            do_task(si_h, di_h, xi_h, np_u, o_sum_user, 0)
            do_task(si_h, di_h, xi_h, np_u, o_sum_user, 1)

        @pl.when(cid == 1)
        def _():
            do_counts(du_h, np_i, o_cnt_item)
            do_task(su_h, du_h, xu_h, np_i, o_sum_item, 2)
            do_task(su_h, du_h, xu_h, np_i, o_sum_item, 3)
            do_task(si_h, di_h, xi_h, np_u, o_sum_user, 2)
            do_task(si_h, di_h, xi_h, np_u, o_sum_user, 3)

    f = pl.kernel(
        body,
        out_type=[
            jax.ShapeDtypeStruct((np_i, d), jnp.float32),
            jax.ShapeDtypeStruct((np_u, d), jnp.float32),
            jax.ShapeDtypeStruct((np_i, CW_CNT), jnp.float32),
            jax.ShapeDtypeStruct((np_u, CW_CNT), jnp.float32),
        ],
        mesh=mesh,
        scratch_types=[
            pltpu.VMEM_SHARED((np_u, cw), jnp.float32),
            pltpu.VMEM((NBUF, chunk), jnp.int32),
            pltpu.VMEM((NBUF, chunk), jnp.int32),
            pltpu.VMEM((NBUF, chunk, cw), jnp.float32),
            pltpu.VMEM((chunk, cw), jnp.float32),
            pltpu.SemaphoreType.DMA((NBUF,)),
            pltpu.SemaphoreType.DMA((NBUF,)),
            pltpu.SemaphoreType.DMA((NBUF,)),
            pltpu.SemaphoreType.DMA((NBUF,)),
        ],
        compiler_params=pltpu.CompilerParams(use_tc_tiling_on_sc=False),
        interpret=interpret,
    )
    return f(xu_flat, xi_flat, su, du, si, di, ones_h, zeros_h)


def _conv_tc(x, ssum, cnt, w_self, w_neigh, bias, interpret=False):
    n, d = x.shape
    br = 1000
    while n % br != 0 or br % 8 != 0:
        br //= 2
    grid = (n // br,)

    def tc_body(x_r, s_r, c_r, ws_r, wn_r, b_r, o_r):
        inv = 1.0 / jnp.maximum(c_r[:, 0:1], 1.0)
        agg = s_r[...] * inv
        o_r[...] = (jnp.dot(x_r[...], ws_r[...], preferred_element_type=jnp.float32)
                    + jnp.dot(agg, wn_r[...], preferred_element_type=jnp.float32)
                    + b_r[...])

    return pl.pallas_call(
        tc_body,
        grid=grid,
        in_specs=[
            pl.BlockSpec((br, d), lambda i: (i, 0)),
            pl.BlockSpec((br, d), lambda i: (i, 0)),
            pl.BlockSpec((br, CW_CNT), lambda i: (i, 0)),
            pl.BlockSpec((d, d), lambda i: (0, 0)),
            pl.BlockSpec((d, d), lambda i: (0, 0)),
            pl.BlockSpec((1, d), lambda i: (0, 0)),
        ],
        out_specs=pl.BlockSpec((br, d), lambda i: (i, 0)),
        out_shape=jax.ShapeDtypeStruct((n, d), jnp.float32),
        interpret=interpret,
    )(x, ssum, cnt, w_self, w_neigh, bias.reshape(1, d))


def _run(x_user, x_item, src_u2i, dst_u2i, src_i2u, dst_i2u,
         w_self_u2i, w_neigh_u2i, b_u2i, w_self_i2u, w_neigh_i2u, b_i2u,
         chunk=80, zrows=136, interpret=False):
    su = src_u2i.astype(jnp.int32)
    du = dst_u2i.astype(jnp.int32)
    si = src_i2u.astype(jnp.int32)
    di = dst_i2u.astype(jnp.int32)
    sum_item, sum_user, cnt_item, cnt_user = _sc_gather_segsum(
        x_user, x_item, su, du, si, di, chunk, zrows, interpret=interpret)
    out_item = _conv_tc(x_item, sum_item, cnt_item, w_self_u2i, w_neigh_u2i,
                        b_u2i, interpret=interpret)
    out_user = _conv_tc(x_user, sum_user, cnt_user, w_self_i2u, w_neigh_i2u,
                        b_i2u, interpret=interpret)
    return (out_item, out_user)


def kernel(x_user, x_item, src_u2i, dst_u2i, src_i2u, dst_i2u,
           W_self_u2i, W_neigh_u2i, b_u2i, W_self_i2u, W_neigh_i2u, b_i2u):
    return _run(x_user, x_item, src_u2i, dst_u2i, src_i2u, dst_i2u,
                W_self_u2i, W_neigh_u2i, b_u2i, W_self_i2u, W_neigh_i2u, b_i2u)


# speedup vs baseline: 1.1674x; 1.1674x over previous
import jax
import jax.numpy as jnp
from jax import lax
from jax.experimental import pallas as pl
from jax.experimental.pallas import tpu as pltpu
from jax.experimental.pallas import tpu_sc as plsc

NS = 16
NC = 2
CC = 4
NBUF = 5
CW_CNT = 16


def _pad_rows(n):
    g = NS * 8
    return (n + g - 1) // g * g


def _sc_gather_segsum(x_user, x_item, su, du, si, di, chunk, zrows,
                      interpret=False):
    n_user, d = x_user.shape
    n_item = x_item.shape[0]
    e = su.shape[0]
    cw = d // CC
    eper = e // NS
    steps = eper // (chunk * NBUF)
    assert eper % (chunk * NBUF) == 0 and chunk % 16 == 0 and chunk <= 128
    np_u, np_i = _pad_rows(n_user), _pad_rows(n_item)
    assert zrows % 8 == 0

    xu_flat = x_user.reshape(n_user * CC, cw)
    xi_flat = x_item.reshape(n_item * CC, cw)
    ones_h = jnp.ones((chunk, cw), jnp.float32)
    zeros_h = jnp.zeros((zrows, cw), jnp.float32)

    mesh = plsc.VectorSubcoreMesh(core_axis_name="c", subcore_axis_name="s",
                                  num_cores=NC, num_subcores=NS)

    def body(xu_h, xi_h, su_h, du_h, si_h, di_h, ones_hb, zeros_hb,
             o_sum_item, o_sum_user, o_cnt_item, o_cnt_user,
             acc, sidx, didx, rows, ones_v, zbuf,
             isem, dsem, gsem, asem):
        cid = lax.axis_index("c")
        sid = lax.axis_index("s")
        pltpu.sync_copy(ones_hb, ones_v)
        pltpu.sync_copy(zeros_hb, zbuf)
        base = sid * eper

        def zero_acc(npad):
            npt = npad // NS
            full, tail = npt // zrows, npt % zrows
            for j in range(full):
                pltpu.sync_copy(zbuf, acc.at[pl.ds(sid * npt + j * zrows, zrows)])
            if tail:
                pltpu.sync_copy(zbuf.at[pl.ds(0, tail)],
                                acc.at[pl.ds(sid * npt + full * zrows, tail)])

        def scan_edges(dst_h, src_h=None, table=None, c=0):
            def issue_idx(i, b):
                off = pl.multiple_of(base + (i * NBUF + b) * chunk, 8)
                dd = pltpu.async_copy(dst_h.at[pl.ds(off, chunk)],
                                      didx.at[b], dsem.at[b])
                sd = None
                if src_h is not None:
                    sd = pltpu.async_copy(src_h.at[pl.ds(off, chunk)],
                                          sidx.at[b], isem.at[b])
                return dd, sd

            def gather(b, sd):
                sd.wait()
                for j in range(chunk // 16):
                    s16 = sidx[b, pl.ds(16 * j, 16)]
                    sidx[b, pl.ds(16 * j, 16)] = s16 * CC + c
                return pltpu.async_copy(table.at[sidx.at[b]],
                                        rows.at[b], gsem.at[b])

            def scatter(b, dd, gd):
                dd.wait()
                if src_h is not None:
                    gd.wait()
                    return pltpu.async_copy(rows.at[b], acc.at[didx.at[b]],
                                            asem.at[b], add=True)
                return pltpu.async_copy(ones_v, acc.at[didx.at[b]],
                                        asem.at[b], add=True)

            def drain_scatter(b):
                if src_h is not None:
                    pltpu.make_async_copy(rows.at[b], acc.at[didx.at[b]],
                                          asem.at[b]).wait()
                else:
                    pltpu.make_async_copy(ones_v, acc.at[didx.at[b]],
                                          asem.at[b]).wait()

            def run_step(i, drain_first):
                dd = [None] * NBUF
                sd = [None] * NBUF
                gd = [None] * NBUF
                for b in range(NBUF):
                    if drain_first:
                        drain_scatter(b)
                    dd[b], sd[b] = issue_idx(i, b)
                if src_h is not None:
                    for b in range(NBUF):
                        gd[b] = gather(b, sd[b])
                for b in range(NBUF):
                    scatter(b, dd[b], gd[b])

            run_step(0, drain_first=False)

            def step(i, carry):
                run_step(i, drain_first=True)
                return carry
            lax.fori_loop(1, steps, step, 0)
            for b in range(NBUF):
                drain_scatter(b)

        def flush(npad, dst):
            npt = npad // NS
            pltpu.sync_copy(acc.at[pl.ds(sid * npt, npt)], dst)

        def do_counts(dst_h, npad, out_ref):
            zero_acc(npad)
            plsc.subcore_barrier()
            scan_edges(dst_h)
            plsc.subcore_barrier()
            npt = npad // NS
            pltpu.sync_copy(acc.at[pl.ds(sid * npt, npt), pl.ds(0, CW_CNT)],
                            out_ref.at[pl.ds(sid * npt, npt)])
            plsc.subcore_barrier()

        def do_task(src_h, dst_h, table, npad, out_ref, c):
            zero_acc(npad)
            plsc.subcore_barrier()
            scan_edges(dst_h, src_h, table, c)
            plsc.subcore_barrier()
            npt = npad // NS
            flush(npad, out_ref.at[pl.ds(sid * npt, npt), pl.ds(cw * c, cw)])
            plsc.subcore_barrier()

        @pl.when(cid == 0)
        def _():
            do_counts(di_h, np_u, o_cnt_user)
            do_task(su_h, du_h, xu_h, np_i, o_sum_item, 0)
            do_task(su_h, du_h, xu_h, np_i, o_sum_item, 1)
            do_task(si_h, di_h, xi_h, np_u, o_sum_user, 0)
            do_task(si_h, di_h, xi_h, np_u, o_sum_user, 1)

        @pl.when(cid == 1)
        def _():
            do_counts(du_h, np_i, o_cnt_item)
            do_task(su_h, du_h, xu_h, np_i, o_sum_item, 2)
            do_task(su_h, du_h, xu_h, np_i, o_sum_item, 3)
            do_task(si_h, di_h, xi_h, np_u, o_sum_user, 2)
            do_task(si_h, di_h, xi_h, np_u, o_sum_user, 3)

    f = pl.kernel(
        body,
        out_type=[
            jax.ShapeDtypeStruct((np_i, d), jnp.float32),
            jax.ShapeDtypeStruct((np_u, d), jnp.float32),
            jax.ShapeDtypeStruct((np_i, CW_CNT), jnp.float32),
            jax.ShapeDtypeStruct((np_u, CW_CNT), jnp.float32),
        ],
        mesh=mesh,
        scratch_types=[
            pltpu.VMEM_SHARED((np_u, cw), jnp.float32),
            pltpu.VMEM((NBUF, chunk), jnp.int32),
            pltpu.VMEM((NBUF, chunk), jnp.int32),
            pltpu.VMEM((NBUF, chunk, cw), jnp.float32),
            pltpu.VMEM((chunk, cw), jnp.float32),
            pltpu.VMEM((zrows, cw), jnp.float32),
            pltpu.SemaphoreType.DMA((NBUF,)),
            pltpu.SemaphoreType.DMA((NBUF,)),
            pltpu.SemaphoreType.DMA((NBUF,)),
            pltpu.SemaphoreType.DMA((NBUF,)),
        ],
        compiler_params=pltpu.CompilerParams(use_tc_tiling_on_sc=False),
        interpret=interpret,
    )
    return f(xu_flat, xi_flat, su, du, si, di, ones_h, zeros_h)


def _conv_tc(x, ssum, cnt, w_self, w_neigh, bias, interpret=False):
    n, d = x.shape
    br = 1000
    while n % br != 0 or br % 8 != 0:
        br //= 2
    grid = (n // br,)

    def tc_body(x_r, s_r, c_r, ws_r, wn_r, b_r, o_r):
        inv = 1.0 / jnp.maximum(c_r[:, 0:1], 1.0)
        agg = s_r[...] * inv
        o_r[...] = (jnp.dot(x_r[...], ws_r[...], preferred_element_type=jnp.float32)
                    + jnp.dot(agg, wn_r[...], preferred_element_type=jnp.float32)
                    + b_r[...])

    return pl.pallas_call(
        tc_body,
        grid=grid,
        in_specs=[
            pl.BlockSpec((br, d), lambda i: (i, 0)),
            pl.BlockSpec((br, d), lambda i: (i, 0)),
            pl.BlockSpec((br, CW_CNT), lambda i: (i, 0)),
            pl.BlockSpec((d, d), lambda i: (0, 0)),
            pl.BlockSpec((d, d), lambda i: (0, 0)),
            pl.BlockSpec((1, d), lambda i: (0, 0)),
        ],
        out_specs=pl.BlockSpec((br, d), lambda i: (i, 0)),
        out_shape=jax.ShapeDtypeStruct((n, d), jnp.float32),
        interpret=interpret,
    )(x, ssum, cnt, w_self, w_neigh, bias.reshape(1, d))


def _run(x_user, x_item, src_u2i, dst_u2i, src_i2u, dst_i2u,
         w_self_u2i, w_neigh_u2i, b_u2i, w_self_i2u, w_neigh_i2u, b_i2u,
         chunk=80, zrows=136, interpret=False):
    su = src_u2i.astype(jnp.int32)
    du = dst_u2i.astype(jnp.int32)
    si = src_i2u.astype(jnp.int32)
    di = dst_i2u.astype(jnp.int32)
    sum_item, sum_user, cnt_item, cnt_user = _sc_gather_segsum(
        x_user, x_item, su, du, si, di, chunk, zrows, interpret=interpret)
    out_item = _conv_tc(x_item, sum_item, cnt_item, w_self_u2i, w_neigh_u2i,
                        b_u2i, interpret=interpret)
    out_user = _conv_tc(x_user, sum_user, cnt_user, w_self_i2u, w_neigh_i2u,
                        b_i2u, interpret=interpret)
    return (out_item, out_user)


def kernel(x_user, x_item, src_u2i, dst_u2i, src_i2u, dst_i2u,
           W_self_u2i, W_neigh_u2i, b_u2i, W_self_i2u, W_neigh_i2u, b_i2u):
    return _run(x_user, x_item, src_u2i, dst_u2i, src_i2u, dst_i2u,
                W_self_u2i, W_neigh_u2i, b_u2i, W_self_i2u, W_neigh_i2u, b_i2u)
